# baseline (speedup 1.0000x reference)
import functools

import jax
import jax.numpy as jnp
import numpy as np
from jax import lax
from jax.experimental import pallas as pl
from jax.experimental.pallas import tpu as pltpu
from jax.experimental.pallas import tpu_sc as plsc

B = 32
PP = 576
L = 16
R = 48
NCHUNK = PP // R
CPR = PP // L
DENOM = float(B * (PP * PP - PP))

C_OUT = np.float32(0.8)
ONE = np.float32(1.0)
FZERO = np.float32(0.0)


def _body(real_hbm, fake_hbm, fw_hbm, ab_hbm, out_hbm,
          rb0, rb1, fb0, fb1, ab0, ab1, fwb, accb,
          sr0, sr1, sf0, sf1, sa0, sa1):
    info = plsc.get_sparse_core_info()
    nc = info.num_cores
    wid = lax.axis_index("s") * nc + lax.axis_index("c")
    base = wid * (PP * PP)
    ab_base = wid * (PP * 2 * L)

    pltpu.sync_copy(fw_hbm.at[pl.ds(wid * PP, PP)], fwb)
    g = []
    for c in range(CPR):
        w = fwb[pl.ds(c * L, L)]
        g.append(jnp.where(w > FZERO, ONE, FZERO))

    rbufs = (rb0, rb1)
    fbufs = (fb0, fb1)
    abufs = (ab0, ab1)
    rsems = (sr0, sr1)
    fsems = (sf0, sf1)
    asems = (sa0, sa1)

    def start(ci):
        k = ci % 2
        off = base + ci * R * PP
        aoff = ab_base + ci * R * 2 * L
        cr = pltpu.async_copy(real_hbm.at[pl.ds(off, R * PP)], rbufs[k], rsems[k])
        cf = pltpu.async_copy(fake_hbm.at[pl.ds(off, R * PP)], fbufs[k], fsems[k])
        ca = pltpu.async_copy(ab_hbm.at[pl.ds(aoff, R * 2 * L)], abufs[k], asems[k])
        return cr, cf, ca

    copies = [None] * NCHUNK
    copies[0] = start(0)

    zero = jnp.zeros((L,), jnp.float32)
    accs = (zero, zero, zero, zero)
    iota = lax.broadcasted_iota(jnp.int32, (L,), 0)

    for ci in range(NCHUNK):
        if ci + 1 < NCHUNK:
            copies[ci + 1] = start(ci + 1)
        for cp in copies[ci]:
            cp.wait()
        rb = rbufs[ci % 2]
        fb = fbufs[ci % 2]
        ab = abufs[ci % 2]

        def rbody(v, a):
            a0, a1, a2, a3 = a
            o = v * (4 * L)
            a0 = a0 + jnp.abs(rb[pl.ds(o, L)] - C_OUT)
            a1 = a1 + jnp.abs(rb[pl.ds(o + L, L)] - C_OUT)
            a2 = a2 + jnp.abs(rb[pl.ds(o + 2 * L, L)] - C_OUT)
            a3 = a3 + jnp.abs(rb[pl.ds(o + 3 * L, L)] - C_OUT)
            return (a0, a1, a2, a3)

        accs = lax.fori_loop(0, R * CPR // 4, rbody, accs)

        def fbody(r, a):
            a0, a1, a2, a3 = a
            i_glob = ci * R + r
            ro = r * PP

            lane = lax.rem(i_glob, L)
            doff = ro + i_glob - lane
            dv = rb[pl.ds(doff, L)]
            corr = jnp.abs(dv - ONE) - jnp.abs(dv - C_OUT)
            a0 = a0 + jnp.where(iota == lane, corr, FZERO)

            av = ab[pl.ds(r * 2 * L, L)]
            bv = ab[pl.ds(r * 2 * L + L, L)]
            aa = [a0, a1, a2, a3]
            for c in range(CPR):
                x = fb[pl.ds(ro + c * L, L)]
                t = av + bv * g[c]
                aa[c % 4] = aa[c % 4] + jnp.abs(x - t)
            return (aa[0], aa[1], aa[2], aa[3])

        accs = lax.fori_loop(0, R, fbody, accs)

    acc = (accs[0] + accs[1]) + (accs[2] + accs[3])
    accb[...] = acc
    pltpu.sync_copy(accb, out_hbm.at[wid])


@jax.jit
def _run(real_flat, fake_flat, fw_flat, ab_flat):
    mesh = plsc.VectorSubcoreMesh(core_axis_name="c", subcore_axis_name="s")
    kfn = functools.partial(
        pl.kernel,
        mesh=mesh,
        out_type=jax.ShapeDtypeStruct((B, L), jnp.float32),
        scratch_types=[
            pltpu.VMEM((R * PP,), jnp.float32),
            pltpu.VMEM((R * PP,), jnp.float32),
            pltpu.VMEM((R * PP,), jnp.float32),
            pltpu.VMEM((R * PP,), jnp.float32),
            pltpu.VMEM((R * 2 * L,), jnp.float32),
            pltpu.VMEM((R * 2 * L,), jnp.float32),
            pltpu.VMEM((PP,), jnp.float32),
            pltpu.VMEM((L,), jnp.float32),
            pltpu.SemaphoreType.DMA,
            pltpu.SemaphoreType.DMA,
            pltpu.SemaphoreType.DMA,
            pltpu.SemaphoreType.DMA,
            pltpu.SemaphoreType.DMA,
            pltpu.SemaphoreType.DMA,
        ],
    )(_body)
    parts = kfn(real_flat, fake_flat, fw_flat, ab_flat)
    return jnp.sum(parts) / np.float32(DENOM)


def kernel(correlation_map_real, correlation_map_fake, fake_weight):
    real_flat = correlation_map_real.reshape(-1)
    fake_flat = correlation_map_fake.reshape(-1)
    fw_flat = fake_weight.reshape(-1)
    fm = fake_weight > 0.0
    a_rows = jnp.where(fm, np.float32(0.1), np.float32(0.8))
    b_rows = jnp.where(fm, np.float32(0.8), np.float32(-0.7))
    ab = jnp.stack([a_rows, b_rows], axis=-1)
    ab = jnp.broadcast_to(ab[..., None], (B, PP, 2, L))
    return _run(real_flat, fake_flat, fw_flat, ab.reshape(-1))


# speedup vs baseline: 3.3015x; 3.3015x over previous
import functools

import jax
import jax.numpy as jnp
import numpy as np
from jax import lax
from jax.experimental import pallas as pl
from jax.experimental.pallas import tpu as pltpu
from jax.experimental.pallas import tpu_sc as plsc

B = 32
PP = 576
L = 16
R = 32
NCHUNK = PP // R
CPR = PP // L
GR = 8
DENOM = float(B * (PP * PP - PP))

C_OUT = np.float32(0.8)
ONE = np.float32(1.0)
FZERO = np.float32(0.0)
A0 = np.float32(0.8)
A1 = np.float32(-0.7)
B0 = np.float32(-0.7)
B1 = np.float32(1.5)

_BCAST = np.kron(np.eye(GR, dtype=np.float32), np.ones((1, L), np.float32))


def _body(real_hbm, fake_hbm, fw_hbm, g_hbm, out_hbm,
          rb0, rb1, fb0, fb1, gball, fwb, accb,
          sr0, sr1, sf0, sf1):
    info = plsc.get_sparse_core_info()
    nc = info.num_cores
    wid = lax.axis_index("s") * nc + lax.axis_index("c")

    pltpu.sync_copy(fw_hbm.at[pl.ds(wid * PP, PP)], fwb)
    pltpu.sync_copy(g_hbm.at[pl.ds(wid * (PP // GR), PP // GR)], gball)
    g = []
    for c in range(CPR):
        w = fwb[pl.ds(c * L, L)]
        g.append(jnp.where(w > FZERO, ONE, FZERO))

    rbufs = (rb0, rb1)
    fbufs = (fb0, fb1)
    rsems = (sr0, sr1)
    fsems = (sf0, sf1)

    def start(ci):
        k = ci % 2
        r0 = ci * R
        cr = pltpu.async_copy(real_hbm.at[wid, pl.ds(r0, R)], rbufs[k], rsems[k])
        cf = pltpu.async_copy(fake_hbm.at[wid, pl.ds(r0, R)], fbufs[k], fsems[k])
        return cr, cf

    copies = [None] * NCHUNK
    copies[0] = start(0)

    zero = jnp.zeros((L,), jnp.float32)
    accs = (zero, zero, zero, zero)
    iota = lax.broadcasted_iota(jnp.int32, (L,), 0)

    for ci in range(NCHUNK):
        if ci + 1 < NCHUNK:
            copies[ci + 1] = start(ci + 1)
        for cp in copies[ci]:
            cp.wait()
        rb = rbufs[ci % 2]
        fb = fbufs[ci % 2]

        def rbody(r, a):
            aa = [a[0], a[1], a[2], a[3]]
            for c in range(CPR):
                aa[c % 4] = aa[c % 4] + jnp.abs(rb[r, pl.ds(c * L, L)] - C_OUT)
            return (aa[0], aa[1], aa[2], aa[3])

        accs = lax.fori_loop(0, R, rbody, accs)

        def fbody(r, a):
            a0, a1, a2, a3 = a
            i_glob = ci * R + r

            lane = lax.rem(i_glob, L)
            dv = rb[r, pl.ds(pl.multiple_of(i_glob - lane, L), L)]
            corr = jnp.abs(dv - ONE) - jnp.abs(dv - C_OUT)
            a0 = a0 + jnp.where(iota == lane, corr, FZERO)

            gi = gball[lax.div(i_glob, GR),
                       pl.ds(pl.multiple_of(lax.rem(i_glob, GR) * L, L), L)]
            av = A0 + A1 * gi
            bv = B0 + B1 * gi
            aa = [a0, a1, a2, a3]
            for c in range(CPR):
                x = fb[r, pl.ds(c * L, L)]
                t = av + bv * g[c]
                aa[c % 4] = aa[c % 4] + jnp.abs(x - t)
            return (aa[0], aa[1], aa[2], aa[3])

        accs = lax.fori_loop(0, R, fbody, accs)

    acc = (accs[0] + accs[1]) + (accs[2] + accs[3])
    accb[...] = acc
    pltpu.sync_copy(accb, out_hbm.at[wid])


@jax.jit
def _run(real, fake, fw_flat, g16):
    mesh = plsc.VectorSubcoreMesh(core_axis_name="c", subcore_axis_name="s")
    kfn = functools.partial(
        pl.kernel,
        mesh=mesh,
        out_type=jax.ShapeDtypeStruct((B, L), jnp.float32),
        scratch_types=[
            pltpu.VMEM((R, PP), jnp.float32),
            pltpu.VMEM((R, PP), jnp.float32),
            pltpu.VMEM((R, PP), jnp.float32),
            pltpu.VMEM((R, PP), jnp.float32),
            pltpu.VMEM((PP // GR, GR * L), jnp.float32),
            pltpu.VMEM((PP,), jnp.float32),
            pltpu.VMEM((L,), jnp.float32),
            pltpu.SemaphoreType.DMA,
            pltpu.SemaphoreType.DMA,
            pltpu.SemaphoreType.DMA,
            pltpu.SemaphoreType.DMA,
        ],
    )(_body)
    parts = kfn(real, fake, fw_flat, g16)
    return jnp.sum(parts) / np.float32(DENOM)


def kernel(correlation_map_real, correlation_map_fake, fake_weight):
    fw_flat = fake_weight.reshape(-1)
    gvals = (fake_weight > 0.0).astype(jnp.float32)
    g16 = gvals.reshape(B * PP // GR, GR) @ jnp.asarray(_BCAST)
    return _run(correlation_map_real, correlation_map_fake, fw_flat, g16)


# speedup vs baseline: 4.5689x; 1.3839x over previous
import functools

import jax
import jax.numpy as jnp
import numpy as np
from jax import lax
from jax.experimental import pallas as pl
from jax.experimental.pallas import tpu as pltpu
from jax.experimental.pallas import tpu_sc as plsc

B = 32
PP = 576
L = 16
R = 32
NCHUNK = PP // R
CPR = PP // L
GR = 8
DENOM = float(B * (PP * PP - PP))

C_OUT = np.float32(0.8)
ONE = np.float32(1.0)
FZERO = np.float32(0.0)
A0 = np.float32(0.8)
A1 = np.float32(-0.7)
B0 = np.float32(-0.7)
B1 = np.float32(1.5)

_BCAST = np.kron(np.eye(GR, dtype=np.float32), np.ones((1, L), np.float32))


def _body(real_hbm, fake_hbm, fw_hbm, g_hbm, out_hbm,
          rb0, rb1, fb0, fb1, gball, fwb, accb,
          sr0, sr1, sf0, sf1):
    info = plsc.get_sparse_core_info()
    nc = info.num_cores
    wid = lax.axis_index("s") * nc + lax.axis_index("c")

    pltpu.sync_copy(fw_hbm.at[pl.ds(wid * PP, PP)], fwb)
    pltpu.sync_copy(g_hbm.at[pl.ds(wid * (PP // GR), PP // GR)], gball)
    g = []
    for c in range(CPR):
        w = fwb[pl.ds(c * L, L)]
        g.append(jnp.where(w > FZERO, ONE, FZERO))

    rbufs = (rb0, rb1)
    fbufs = (fb0, fb1)
    rsems = (sr0, sr1)
    fsems = (sf0, sf1)

    def start(ci):
        k = ci % 2
        r0 = ci * R
        cr = pltpu.async_copy(real_hbm.at[wid, pl.ds(r0, R)], rbufs[k], rsems[k])
        cf = pltpu.async_copy(fake_hbm.at[wid, pl.ds(r0, R)], fbufs[k], fsems[k])
        return cr, cf

    copies = [None] * NCHUNK
    copies[0] = start(0)

    zero = jnp.zeros((L,), jnp.float32)
    accs = (zero, zero, zero, zero)
    iota = lax.broadcasted_iota(jnp.int32, (L,), 0)

    for ci in range(NCHUNK):
        if ci + 1 < NCHUNK:
            copies[ci + 1] = start(ci + 1)
        for cp in copies[ci]:
            cp.wait()
        rb = rbufs[ci % 2]
        fb = fbufs[ci % 2]

        def rbody(r, a):
            aa = [a[0], a[1], a[2], a[3]]
            for c in range(CPR):
                aa[c % 4] = aa[c % 4] + jnp.abs(rb[r, pl.ds(c * L, L)] - C_OUT)
            return (aa[0], aa[1], aa[2], aa[3])

        accs = lax.fori_loop(0, R, rbody, accs)

        def fbody(r, a):
            a0, a1, a2, a3 = a
            i_glob = ci * R + r

            lane = lax.rem(i_glob, L)
            dv = rb[r, pl.ds(pl.multiple_of(i_glob - lane, L), L)]
            corr = jnp.abs(dv - ONE) - jnp.abs(dv - C_OUT)
            a0 = a0 + jnp.where(iota == lane, corr, FZERO)

            gi = gball[lax.div(i_glob, GR),
                       pl.ds(pl.multiple_of(lax.rem(i_glob, GR) * L, L), L)]
            av = A0 + A1 * gi
            bv = B0 + B1 * gi
            aa = [a0, a1, a2, a3]
            for c in range(CPR):
                x = fb[r, pl.ds(c * L, L)]
                t = av + bv * g[c]
                aa[c % 4] = aa[c % 4] + jnp.abs(x - t)
            return (aa[0], aa[1], aa[2], aa[3])

        accs = lax.fori_loop(0, R, fbody, accs)

    acc = (accs[0] + accs[1]) + (accs[2] + accs[3])
    accb[...] = acc
    pltpu.sync_copy(accb, out_hbm.at[wid])


@jax.jit
def _run(real, fake, fw_flat, g16):
    mesh = plsc.VectorSubcoreMesh(core_axis_name="c", subcore_axis_name="s")
    kfn = functools.partial(
        pl.kernel,
        mesh=mesh,
        out_type=jax.ShapeDtypeStruct((B, L), jnp.float32),
        scratch_types=[
            pltpu.VMEM((R, PP), jnp.float32),
            pltpu.VMEM((R, PP), jnp.float32),
            pltpu.VMEM((R, PP), jnp.float32),
            pltpu.VMEM((R, PP), jnp.float32),
            pltpu.VMEM((PP // GR, GR * L), jnp.float32),
            pltpu.VMEM((PP,), jnp.float32),
            pltpu.VMEM((L,), jnp.float32),
            pltpu.SemaphoreType.DMA,
            pltpu.SemaphoreType.DMA,
            pltpu.SemaphoreType.DMA,
            pltpu.SemaphoreType.DMA,
        ],
    )(_body)
    parts = kfn(real, fake, fw_flat, g16)
    return jnp.sum(parts) / np.float32(DENOM)


def _tc_body(grow_ref, gcol_ref, real_ref, fake_ref, out_ref):
    b = pl.program_id(0)

    ii = lax.broadcasted_iota(jnp.int32, (PP, PP), 0)
    jj = lax.broadcasted_iota(jnp.int32, (PP, PP), 1)
    x = real_ref[0]
    tr = jnp.where(ii == jj, ONE, C_OUT)
    s_real = jnp.sum(jnp.abs(x - tr))

    y = fake_ref[0]
    gj = grow_ref[0]
    gi = gcol_ref[0]
    t = A0 + A1 * (gi + gj) + np.float32(1.5) * (gi * gj)
    s_fake = jnp.sum(jnp.abs(y - t))

    @pl.when(b == 0)
    def _init():
        out_ref[0, 0] = FZERO

    out_ref[0, 0] += (s_real + s_fake) * np.float32(1.0 / DENOM)


@jax.jit
def _run_tc(real, fake, fw):
    g = jnp.where(fw > 0.0, np.float32(1.0), np.float32(0.0))
    grow = g[:, None, :]
    gcol = g[:, :, None]
    out = pl.pallas_call(
        _tc_body,
        grid=(B,),
        in_specs=[
            pl.BlockSpec((1, 1, PP), lambda i: (i, 0, 0)),
            pl.BlockSpec((1, PP, 1), lambda i: (i, 0, 0)),
            pl.BlockSpec((1, PP, PP), lambda i: (i, 0, 0)),
            pl.BlockSpec((1, PP, PP), lambda i: (i, 0, 0)),
        ],
        out_specs=pl.BlockSpec(
            (1, 1), lambda i: (0, 0), memory_space=pltpu.SMEM),
        out_shape=jax.ShapeDtypeStruct((1, 1), jnp.float32),
    )(grow, gcol, real, fake)
    return out.reshape(())


def kernel(correlation_map_real, correlation_map_fake, fake_weight):
    return _run_tc(correlation_map_real, correlation_map_fake, fake_weight)


def _kernel_sc(correlation_map_real, correlation_map_fake, fake_weight):
    fw_flat = fake_weight.reshape(-1)
    gvals = (fake_weight > 0.0).astype(jnp.float32)
    g16 = gvals.reshape(B * PP // GR, GR) @ jnp.asarray(_BCAST)
    return _run(correlation_map_real, correlation_map_fake, fw_flat, g16)
